# baseline (speedup 1.0000x reference)
import jax
import jax.numpy as jnp
from jax import lax
from jax.experimental import pallas as pl
from jax.experimental.pallas import tpu as pltpu
from jax.experimental.pallas import tpu_sc as plsc

NUM_EMBEDDINGS = 100000
DIM = 128
BATCH = 16384

NC = 2
NS = 16
L = 16
NW = NC * NS
B_PER_W = BATCH // NW
G_CHUNK = 128
N_G = B_PER_W // G_CHUNK
C_PER_ROW = DIM // L


def _rsqrt_nr(s):
    i = lax.bitcast_convert_type(s, jnp.int32)
    y = lax.bitcast_convert_type(jnp.int32(0x5F3759DF) - (i >> 1), jnp.float32)
    for _ in range(1):
        y = y * (1.5 - 0.5 * s * y * y)
    return y


def _sc_lookup_normalize(task_id, table):
    mesh = plsc.VectorSubcoreMesh(core_axis_name="c", subcore_axis_name="s")

    @pl.kernel(
        out_type=jax.ShapeDtypeStruct((BATCH, DIM), jnp.float32),
        mesh=mesh,
        scratch_types=[
            pltpu.VMEM((B_PER_W,), jnp.int32),
            pltpu.VMEM((B_PER_W, DIM), jnp.float32),
            pltpu.SemaphoreType.DMA,
            pltpu.SemaphoreType.DMA,
        ],
    )
    def k(idx_hbm, tab_hbm, out_hbm, idx_v, rows_v, gsem, wsem):
        wid = lax.axis_index("s") * NC + lax.axis_index("c")
        base = wid * B_PER_W

        def gather(j):
            return pltpu.make_async_copy(
                tab_hbm.at[idx_v.at[pl.ds(j * G_CHUNK, G_CHUNK)]],
                rows_v.at[pl.ds(j * G_CHUNK, G_CHUNK)],
                gsem,
            )

        def write(j):
            return pltpu.make_async_copy(
                rows_v.at[pl.ds(j * G_CHUNK, G_CHUNK)],
                out_hbm.at[pl.ds(base + j * G_CHUNK, G_CHUNK)],
                wsem,
            )

        pltpu.sync_copy(idx_hbm.at[pl.ds(base, G_CHUNK)], idx_v.at[pl.ds(0, G_CHUNK)])
        gather(0).start()
        pltpu.sync_copy(
            idx_hbm.at[pl.ds(base + G_CHUNK, B_PER_W - G_CHUNK)],
            idx_v.at[pl.ds(G_CHUNK, B_PER_W - G_CHUNK)],
        )

        def fire_body(j, _):
            gather(j).start()
            return _

        lax.fori_loop(1, N_G, fire_body, None)

        def chunk_body(j, _):
            gather(j).wait()

            @plsc.parallel_loop(j * G_CHUNK, (j + 1) * G_CHUNK, unroll=4)
            def _(r):
                xs = [rows_v[r, pl.ds(c * L, L)] for c in range(C_PER_ROW)]
                acc = xs[0] * xs[0]
                for c in range(1, C_PER_ROW):
                    acc = acc + xs[c] * xs[c]
                lane = lax.iota(jnp.int32, L)
                for sh in (8, 4, 2, 1):
                    acc = acc + acc[lane ^ sh]
                s = jnp.maximum(acc, 1e-24)
                y = _rsqrt_nr(s)
                for c in range(C_PER_ROW):
                    rows_v[r, pl.ds(c * L, L)] = xs[c] * y

            return _

        lax.fori_loop(0, N_G, chunk_body, None)

        def write_body(j, _):
            write(j).start()
            return _

        lax.fori_loop(0, N_G, write_body, None)

        pltpu.make_async_copy(
            rows_v, out_hbm.at[pl.ds(base, B_PER_W)], wsem
        ).wait()

    return k(task_id, table)


def kernel(task_id, embedding_weight):
    return _sc_lookup_normalize(task_id.astype(jnp.int32), embedding_weight)


# speedup vs baseline: 1.0701x; 1.0701x over previous
import jax
import jax.numpy as jnp
from jax import lax
from jax.experimental import pallas as pl
from jax.experimental.pallas import tpu as pltpu
from jax.experimental.pallas import tpu_sc as plsc

NUM_EMBEDDINGS = 100000
DIM = 128
BATCH = 16384

NC = 2
NS = 16
L = 16
NW = NC * NS
B_PER_W = BATCH // NW
G_CHUNK = 128
N_G = B_PER_W // G_CHUNK
C_PER_ROW = DIM // L


def _rsqrt_nr(s):
    i = lax.bitcast_convert_type(s, jnp.int32)
    y = lax.bitcast_convert_type(jnp.int32(0x5F3759DF) - (i >> 1), jnp.float32)
    for _ in range(1):
        y = y * (1.5 - 0.5 * s * y * y)
    return y


def _sc_lookup_normalize(task_id, table):
    mesh = plsc.VectorSubcoreMesh(core_axis_name="c", subcore_axis_name="s")

    @pl.kernel(
        out_type=jax.ShapeDtypeStruct((BATCH, DIM), jnp.float32),
        mesh=mesh,
        scratch_types=[
            pltpu.VMEM((B_PER_W,), jnp.int32),
            pltpu.VMEM((B_PER_W, DIM), jnp.float32),
            pltpu.SemaphoreType.DMA,
            pltpu.SemaphoreType.DMA,
        ],
    )
    def k(idx_hbm, tab_hbm, out_hbm, idx_v, rows_v, gsem, wsem):
        wid = lax.axis_index("s") * NC + lax.axis_index("c")
        base = wid * B_PER_W

        def gather(j):
            return pltpu.make_async_copy(
                tab_hbm.at[idx_v.at[pl.ds(j * G_CHUNK, G_CHUNK)]],
                rows_v.at[pl.ds(j * G_CHUNK, G_CHUNK)],
                gsem,
            )

        def write(j):
            return pltpu.make_async_copy(
                rows_v.at[pl.ds(j * G_CHUNK, G_CHUNK)],
                out_hbm.at[pl.ds(base + j * G_CHUNK, G_CHUNK)],
                wsem,
            )

        pltpu.sync_copy(idx_hbm.at[pl.ds(base, G_CHUNK)], idx_v.at[pl.ds(0, G_CHUNK)])
        gather(0).start()
        pltpu.sync_copy(
            idx_hbm.at[pl.ds(base + G_CHUNK, B_PER_W - G_CHUNK)],
            idx_v.at[pl.ds(G_CHUNK, B_PER_W - G_CHUNK)],
        )

        def fire_body(j, _):
            gather(j).start()
            return _

        lax.fori_loop(1, N_G, fire_body, None)

        def chunk_body(j, _):
            gather(j).wait()

            @plsc.parallel_loop(j * G_CHUNK, (j + 1) * G_CHUNK, unroll=4)
            def _(r):
                xs = [rows_v[r, pl.ds(c * L, L)] for c in range(C_PER_ROW)]
                acc = xs[0] * xs[0]
                for c in range(1, C_PER_ROW):
                    acc = acc + xs[c] * xs[c]
                lane = lax.iota(jnp.int32, L)
                for sh in (8, 4, 2, 1):
                    acc = acc + acc[lane ^ sh]
                s = jnp.maximum(acc, 1e-24)
                y = _rsqrt_nr(s)
                for c in range(C_PER_ROW):
                    rows_v[r, pl.ds(c * L, L)] = xs[c] * y

            write(j).start()
            return _

        lax.fori_loop(0, N_G, chunk_body, None)

        pltpu.make_async_copy(
            rows_v, out_hbm.at[pl.ds(base, B_PER_W)], wsem
        ).wait()

    return k(task_id, table)


def kernel(task_id, embedding_weight):
    return _sc_lookup_normalize(task_id.astype(jnp.int32), embedding_weight)
